# Initial kernel scaffold; baseline (speedup 1.0000x reference)
import jax
import jax.numpy as jnp
from jax.experimental import pallas as pl


def kernel(node_attribute, edge_attribute, edge_length, edge_index, segment_ids, proj_W, proj_b, bond_W, bond_b, conv_b, gru_Wih, gru_Whh, gru_bih, gru_bhh, spars_W, spars_b, prelu_a):
    raise NotImplementedError("write your pallas kernel here")


# speedup vs baseline: 1.1100x; 1.1100x over previous
import functools

import jax
import jax.numpy as jnp
from jax import lax
from jax.experimental import pallas as pl
from jax.experimental.pallas import tpu as pltpu
from jax.experimental.pallas import tpu_sc as plsc

N = 10000
E = 160000
G = 64
F = 32
D_EDGE = 8
D_RBF = 8
D_HID = 4096
STEPS = 4

N_PAD = 10240
E_PAD = 163840
CHUNK = 128
N_CHUNKS = E_PAD // CHUNK
NW = 32
CH_PER_W = N_CHUNKS // NW
GRP = 8
N_GRP = CH_PER_W // GRP
ROWS_PER_SUB = N_PAD // 16

EBLK = 2048
HIGH = lax.Precision.HIGHEST

@functools.lru_cache(maxsize=1)
def _build_sc_kernels():
    mesh = plsc.VectorSubcoreMesh(core_axis_name="c", subcore_axis_name="s",
                                  num_cores=2, num_subcores=16)
    params = pltpu.CompilerParams(use_tc_tiling_on_sc=False)

    @functools.partial(
        pl.kernel,
        out_type=jax.ShapeDtypeStruct((E_PAD, F), jnp.float32),
        mesh=mesh,
        compiler_params=params,
        scratch_types=[
            pltpu.VMEM((GRP, CHUNK), jnp.int32),
            pltpu.VMEM((GRP * CHUNK, F), jnp.float32),
            pltpu.SemaphoreType.DMA,
        ],
    )
    def sc_gather(h_hbm, src_hbm, out_hbm, idx_v, rows_v, sem):
        wid = lax.axis_index("s") * 2 + lax.axis_index("c")
        for g in range(N_GRP):
            chunk0 = wid * CH_PER_W + g * GRP
            pltpu.sync_copy(src_hbm.at[pl.ds(chunk0, GRP)], idx_v)
            copies = []
            for j in range(GRP):
                copies.append(
                    pltpu.async_copy(
                        h_hbm.at[idx_v.at[j]],
                        rows_v.at[pl.ds(j * CHUNK, CHUNK)],
                        sem,
                    )
                )
            for c in copies:
                c.wait()
            pltpu.sync_copy(rows_v, out_hbm.at[pl.ds(chunk0 * CHUNK, GRP * CHUNK)])

    @functools.partial(
        pl.kernel,
        out_type=jax.ShapeDtypeStruct((2, N_PAD, F), jnp.float32),
        mesh=mesh,
        compiler_params=params,
        scratch_types=[
            pltpu.VMEM((GRP, CHUNK), jnp.int32),
            pltpu.VMEM((GRP * CHUNK, F), jnp.float32),
            pltpu.VMEM_SHARED((N_PAD, F), jnp.float32),
        ],
    )
    def sc_scatter(m_hbm, dst_hbm, zero_hbm, out_hbm, idx_v, rows_v, agg_sh):
        cid = lax.axis_index("c")
        sid = lax.axis_index("s")
        wid = sid * 2 + cid
        row0 = sid * ROWS_PER_SUB
        pltpu.sync_copy(zero_hbm.at[pl.ds(row0, ROWS_PER_SUB)],
                        agg_sh.at[pl.ds(row0, ROWS_PER_SUB)])
        plsc.subcore_barrier()
        for g in range(N_GRP):
            chunk0 = wid * CH_PER_W + g * GRP
            pltpu.sync_copy(dst_hbm.at[pl.ds(chunk0, GRP)], idx_v)
            pltpu.sync_copy(m_hbm.at[pl.ds(chunk0 * CHUNK, GRP * CHUNK)], rows_v)
            for j in range(GRP):
                pltpu.sync_copy(rows_v.at[pl.ds(j * CHUNK, CHUNK)],
                                agg_sh.at[idx_v.at[j]], add=True)
        plsc.subcore_barrier()
        pltpu.sync_copy(agg_sh.at[pl.ds(row0, ROWS_PER_SUB)],
                        rows_v.at[pl.ds(0, ROWS_PER_SUB)])
        pltpu.sync_copy(rows_v.at[pl.ds(0, ROWS_PER_SUB)],
                        out_hbm.at[cid, pl.ds(row0, ROWS_PER_SUB)])

    return sc_gather, sc_scatter


def _proj_body(na_ref, w_ref, b_ref, out_ref):
    h = jnp.dot(na_ref[...], w_ref[...], preferred_element_type=jnp.float32,
                precision=HIGH)
    out_ref[...] = jnp.maximum(h + b_ref[...], 0.0)


def _ef_body(ea_ref, el_ref, out_ref, *, blk):
    i = pl.program_id(0)
    el = el_ref[...]
    centers = lax.broadcasted_iota(jnp.int32, (1, D_RBF), 1).astype(jnp.float32) / (D_RBF - 1.0)
    rbf = jnp.exp(-10.0 * (el - centers) ** 2)
    ef = jnp.concatenate([ea_ref[...], rbf], axis=1)
    row = i * blk + lax.broadcasted_iota(jnp.int32, (blk, 1), 0)
    out_ref[...] = jnp.where(row < E, ef, 0.0)


def _msg_body(ef_ref, hs_ref, b2_ref, out_ref):
    ef = ef_ref[...]
    hs = hs_ref[...]
    u = jnp.concatenate([ef[:, k:k + 1] * hs for k in range(D_EDGE + D_RBF)],
                        axis=1)
    out_ref[...] = jnp.dot(u, b2_ref[...], preferred_element_type=jnp.float32,
                           precision=HIGH)


def _gru_body(agg_ref, cb_ref, hid_ref, wih_ref, whh_ref, bih_ref, bhh_ref,
              out_ref):
    x = jnp.maximum(agg_ref[0] + agg_ref[1] + cb_ref[...], 0.0)
    hid = hid_ref[...]
    gi = jnp.dot(x, wih_ref[...], preferred_element_type=jnp.float32,
                 precision=HIGH) + bih_ref[...]
    gh = jnp.dot(hid, whh_ref[...], preferred_element_type=jnp.float32,
                 precision=HIGH) + bhh_ref[...]
    r = jax.nn.sigmoid(gi[:, 0:F] + gh[:, 0:F])
    z = jax.nn.sigmoid(gi[:, F:2 * F] + gh[:, F:2 * F])
    n = jnp.tanh(gi[:, 2 * F:] + r * gh[:, 2 * F:])
    out_ref[...] = (1.0 - z) * n + z * hid


def _epi_body(h_ref, h0_ref, seg_ref, w_ref, b_ref, a_ref, out_ref):
    na = jnp.concatenate([h_ref[...], h0_ref[...]], axis=1)
    gids = lax.broadcasted_iota(jnp.int32, (1, G), 1)
    onehot = (seg_ref[...] == gids).astype(jnp.float32)
    r = lax.dot_general(onehot, na, (((0,), (0,)), ((), ())),
                        preferred_element_type=jnp.float32, precision=HIGH)
    y = jnp.dot(r, w_ref[...], preferred_element_type=jnp.float32,
                precision=HIGH) + b_ref[...]
    out_ref[...] = jnp.where(y >= 0.0, y, a_ref[0, 0] * y)


def kernel(node_attribute, edge_attribute, edge_length, edge_index, segment_ids,
           proj_W, proj_b, bond_W, bond_b, conv_b,
           gru_Wih, gru_Whh, gru_bih, gru_bhh,
           spars_W, spars_b, prelu_a):
    f32 = jnp.float32
    na_pad = jnp.zeros((N_PAD, node_attribute.shape[1]), f32).at[:N].set(node_attribute)
    src = jnp.zeros((E_PAD,), jnp.int32).at[:E].set(edge_index[0]).reshape(N_CHUNKS, CHUNK)
    dst = jnp.zeros((E_PAD,), jnp.int32).at[:E].set(edge_index[1]).reshape(N_CHUNKS, CHUNK)
    ea_pad = jnp.zeros((E_PAD, D_EDGE), f32).at[:E].set(edge_attribute)
    el_pad = jnp.zeros((E_PAD, 1), f32).at[:E, 0].set(edge_length)
    seg_pad = jnp.full((N_PAD, 1), G, jnp.int32).at[:N, 0].set(segment_ids)
    b2 = bond_W.reshape(16 * F, F)
    zeros_np = jnp.zeros((N_PAD, F), f32)

    h0 = pl.pallas_call(
        _proj_body,
        out_shape=jax.ShapeDtypeStruct((N_PAD, F), f32),
    )(na_pad, proj_W, proj_b.reshape(1, F))

    n_eblk = E_PAD // EBLK
    ef = pl.pallas_call(
        functools.partial(_ef_body, blk=EBLK),
        grid=(n_eblk,),
        in_specs=[pl.BlockSpec((EBLK, D_EDGE), lambda i: (i, 0)),
                  pl.BlockSpec((EBLK, 1), lambda i: (i, 0))],
        out_specs=pl.BlockSpec((EBLK, D_RBF + D_EDGE), lambda i: (i, 0)),
        out_shape=jax.ShapeDtypeStruct((E_PAD, D_RBF + D_EDGE), f32),
    )(ea_pad, el_pad)

    msg = pl.pallas_call(
        _msg_body,
        grid=(n_eblk,),
        in_specs=[pl.BlockSpec((EBLK, D_RBF + D_EDGE), lambda i: (i, 0)),
                  pl.BlockSpec((EBLK, F), lambda i: (i, 0)),
                  pl.BlockSpec((16 * F, F), lambda i: (0, 0))],
        out_specs=pl.BlockSpec((EBLK, F), lambda i: (i, 0)),
        out_shape=jax.ShapeDtypeStruct((E_PAD, F), f32),
    )

    gru = pl.pallas_call(
        _gru_body,
        out_shape=jax.ShapeDtypeStruct((N_PAD, F), f32),
    )

    sc_gather, sc_scatter = _build_sc_kernels()
    h = h0
    for _ in range(STEPS):
        hs = sc_gather(h, src)
        m = msg(ef, hs, b2)
        aggp = sc_scatter(m, dst, zeros_np)
        h = gru(aggp, conv_b.reshape(1, F), h,
                gru_Wih, gru_Whh, gru_bih.reshape(1, 3 * F),
                gru_bhh.reshape(1, 3 * F))

    out = pl.pallas_call(
        _epi_body,
        out_shape=jax.ShapeDtypeStruct((G, D_HID), f32),
    )(h, h0, seg_pad, spars_W, spars_b.reshape(1, D_HID),
      prelu_a.reshape(1, 1))
    return out


# speedup vs baseline: 2.2605x; 2.0364x over previous
import functools

import jax
import jax.numpy as jnp
from jax import lax
from jax.experimental import pallas as pl
from jax.experimental.pallas import tpu as pltpu
from jax.experimental.pallas import tpu_sc as plsc

N = 10000
E = 160000
G = 64
F = 32
D_EDGE = 8
D_RBF = 8
D_HID = 4096
STEPS = 4

N_PAD = 10240
E_PAD = 163840
CHUNK = 128
N_CHUNKS = E_PAD // CHUNK
NW = 32
CH_PER_W = N_CHUNKS // NW
GRP = 8
N_GRP = CH_PER_W // GRP
ROWS_PER_SUB = N_PAD // 16

EBLK = 2048
HIGH = lax.Precision.HIGHEST

@functools.lru_cache(maxsize=1)
def _build_sc_kernels():
    mesh = plsc.VectorSubcoreMesh(core_axis_name="c", subcore_axis_name="s",
                                  num_cores=2, num_subcores=16)
    params = pltpu.CompilerParams(use_tc_tiling_on_sc=False)

    @functools.partial(
        pl.kernel,
        out_type=jax.ShapeDtypeStruct((E_PAD, F), jnp.float32),
        mesh=mesh,
        compiler_params=params,
        scratch_types=[
            pltpu.VMEM((CH_PER_W, CHUNK), jnp.int32),
            pltpu.VMEM((2, GRP * CHUNK, F), jnp.float32),
            pltpu.SemaphoreType.DMA,
            pltpu.SemaphoreType.DMA,
            pltpu.SemaphoreType.DMA,
            pltpu.SemaphoreType.DMA,
        ],
    )
    def sc_gather(h_hbm, src_hbm, out_hbm, idx_v, rows_v, ga0, ga1, oa0, oa1):
        wid = lax.axis_index("s") * 2 + lax.axis_index("c")
        chunk0 = wid * CH_PER_W
        pltpu.sync_copy(src_hbm.at[pl.ds(chunk0, CH_PER_W)], idx_v)
        gsem = [ga0, ga1]
        osem = [oa0, oa1]
        gath = {}
        outc = {}

        def fire(g):
            b = g % 2
            gath[g] = [
                pltpu.async_copy(h_hbm.at[idx_v.at[g * GRP + j]],
                                 rows_v.at[b, pl.ds(j * CHUNK, CHUNK)],
                                 gsem[b])
                for j in range(GRP)
            ]

        fire(0)
        for g in range(N_GRP):
            b = g % 2
            if g + 1 < N_GRP:
                if g >= 1:
                    outc[g - 1].wait()
                fire(g + 1)
            for c in gath[g]:
                c.wait()
            outc[g] = pltpu.async_copy(
                rows_v.at[b],
                out_hbm.at[pl.ds((chunk0 + g * GRP) * CHUNK, GRP * CHUNK)],
                osem[b])
        outc[N_GRP - 2].wait()
        outc[N_GRP - 1].wait()

    @functools.partial(
        pl.kernel,
        out_type=jax.ShapeDtypeStruct((2, N_PAD, F), jnp.float32),
        mesh=mesh,
        compiler_params=params,
        scratch_types=[
            pltpu.VMEM((CH_PER_W, CHUNK), jnp.int32),
            pltpu.VMEM((2, GRP * CHUNK, F), jnp.float32),
            pltpu.VMEM_SHARED((N_PAD, F), jnp.float32),
            pltpu.SemaphoreType.DMA,
            pltpu.SemaphoreType.DMA,
            pltpu.SemaphoreType.DMA,
            pltpu.SemaphoreType.DMA,
        ],
    )
    def sc_scatter(m_hbm, dst_hbm, zero_hbm, out_hbm, idx_v, rows_v, agg_sh,
                   ma0, ma1, sa0, sa1):
        cid = lax.axis_index("c")
        sid = lax.axis_index("s")
        wid = sid * 2 + cid
        row0 = sid * ROWS_PER_SUB
        chunk0 = wid * CH_PER_W
        zc = pltpu.async_copy(zero_hbm.at[pl.ds(row0, ROWS_PER_SUB)],
                              agg_sh.at[pl.ds(row0, ROWS_PER_SUB)], ma1)
        pltpu.sync_copy(dst_hbm.at[pl.ds(chunk0, CH_PER_W)], idx_v)
        zc.wait()
        plsc.subcore_barrier()
        msem = [ma0, ma1]
        ssem = [sa0, sa1]
        mload = {}
        adds = {}

        def fire_load(g):
            b = g % 2
            mload[g] = pltpu.async_copy(
                m_hbm.at[pl.ds((chunk0 + g * GRP) * CHUNK, GRP * CHUNK)],
                rows_v.at[b], msem[b])

        fire_load(0)
        for g in range(N_GRP):
            b = g % 2
            mload[g].wait()
            if g + 1 < N_GRP:
                if g >= 1:
                    for c in adds[g - 1]:
                        c.wait()
                fire_load(g + 1)
            adds[g] = [
                pltpu.async_copy(rows_v.at[b, pl.ds(j * CHUNK, CHUNK)],
                                 agg_sh.at[idx_v.at[g * GRP + j]],
                                 ssem[b], add=True)
                for j in range(GRP)
            ]
        for g in (N_GRP - 2, N_GRP - 1):
            for c in adds[g]:
                c.wait()
        plsc.subcore_barrier()
        pltpu.sync_copy(agg_sh.at[pl.ds(row0, ROWS_PER_SUB)],
                        rows_v.at[0, pl.ds(0, ROWS_PER_SUB)])
        pltpu.sync_copy(rows_v.at[0, pl.ds(0, ROWS_PER_SUB)],
                        out_hbm.at[cid, pl.ds(row0, ROWS_PER_SUB)])

    return sc_gather, sc_scatter


def _proj_body(na_ref, w_ref, b_ref, out_ref):
    h = jnp.dot(na_ref[...], w_ref[...], preferred_element_type=jnp.float32,
                precision=HIGH)
    out_ref[pl.ds(0, N), :] = jnp.maximum(h + b_ref[...], 0.0)
    out_ref[pl.ds(N, N_PAD - N), :] = jnp.zeros((N_PAD - N, F), jnp.float32)


def _ef_body(ea_ref, el_ref, out_ref, *, blk):
    i = pl.program_id(0)
    el = el_ref[...]
    centers = lax.broadcasted_iota(jnp.int32, (1, D_RBF), 1).astype(jnp.float32) / (D_RBF - 1.0)
    rbf = jnp.exp(-10.0 * (el - centers) ** 2)
    ef = jnp.concatenate([ea_ref[...], rbf], axis=1)
    row = i * blk + lax.broadcasted_iota(jnp.int32, (blk, 1), 0)
    out_ref[...] = jnp.where(row < E, ef, 0.0)


def _bdot(a, b):
    return jax.lax.dot_general(a, b, (((1,), (0,)), ((), ())),
                               preferred_element_type=jnp.float32)


def _split(x):
    hi = x.astype(jnp.bfloat16)
    lo = (x - hi.astype(jnp.float32)).astype(jnp.bfloat16)
    return hi, lo


def _msg_body(ef_ref, hs_ref, bah_ref, bal_ref, out_ref):
    ef = ef_ref[...]
    hs = hs_ref[...]
    hs_h, hs_l = _split(hs)
    bah = bah_ref[...]
    t = (_bdot(hs_h, bah) + _bdot(hs_l, bah)) + _bdot(hs_h, bal_ref[...])
    kidx = lax.broadcasted_iota(jnp.int32, (ef.shape[0], 16 * F), 1) // F
    ef_rep = jnp.take_along_axis(ef, kidx, axis=1)
    x = ef_rep * t
    x = x[:, :256] + x[:, 256:]
    x = x[:, :128] + x[:, 128:]
    x = x[:, :64] + x[:, 64:]
    out_ref[...] = x[:, :F] + x[:, F:]


def _gru_body(agg_ref, cb_ref, hid_ref, wih_ref, whh_ref, bih_ref, bhh_ref,
              out_ref):
    x = jnp.maximum(agg_ref[0] + agg_ref[1] + cb_ref[...], 0.0)
    hid = hid_ref[...]
    gi = jnp.dot(x, wih_ref[...], preferred_element_type=jnp.float32,
                 precision=HIGH) + bih_ref[...]
    gh = jnp.dot(hid, whh_ref[...], preferred_element_type=jnp.float32,
                 precision=HIGH) + bhh_ref[...]
    r = jax.nn.sigmoid(gi[:, 0:F] + gh[:, 0:F])
    z = jax.nn.sigmoid(gi[:, F:2 * F] + gh[:, F:2 * F])
    n = jnp.tanh(gi[:, 2 * F:] + r * gh[:, 2 * F:])
    out_ref[...] = (1.0 - z) * n + z * hid


def _epi_body(h_ref, h0_ref, seg_ref, w_ref, b_ref, a_ref, out_ref):
    na = jnp.concatenate([h_ref[...], h0_ref[...]], axis=1)
    gids = lax.broadcasted_iota(jnp.int32, (1, G), 1)
    onehot = (seg_ref[...] == gids).astype(jnp.float32)
    r = lax.dot_general(onehot, na, (((0,), (0,)), ((), ())),
                        preferred_element_type=jnp.float32, precision=HIGH)
    y = jnp.dot(r, w_ref[...], preferred_element_type=jnp.float32,
                precision=HIGH) + b_ref[...]
    out_ref[...] = jnp.where(y >= 0.0, y, a_ref[0, 0] * y)


def kernel(node_attribute, edge_attribute, edge_length, edge_index, segment_ids,
           proj_W, proj_b, bond_W, bond_b, conv_b,
           gru_Wih, gru_Whh, gru_bih, gru_bhh,
           spars_W, spars_b, prelu_a):
    f32 = jnp.float32
    src = jnp.zeros((E_PAD,), jnp.int32).at[:E].set(edge_index[0]).reshape(N_CHUNKS, CHUNK)
    dst = jnp.zeros((E_PAD,), jnp.int32).at[:E].set(edge_index[1]).reshape(N_CHUNKS, CHUNK)
    seg_pad = jnp.full((N_PAD, 1), G, jnp.int32).at[:N, 0].set(segment_ids)
    b2all = bond_W.reshape(16, F, F).transpose(1, 0, 2).reshape(F, 16 * F)
    bah = b2all.astype(jnp.bfloat16)
    bal = (b2all - bah.astype(f32)).astype(jnp.bfloat16)
    zeros_np = jnp.zeros((N_PAD, F), f32)

    h0 = pl.pallas_call(
        _proj_body,
        out_shape=jax.ShapeDtypeStruct((N_PAD, F), f32),
    )(node_attribute, proj_W, proj_b.reshape(1, F))

    n_eblk = E_PAD // EBLK
    last_in = E // EBLK - (1 if E % EBLK == 0 else 0)
    ef = pl.pallas_call(
        functools.partial(_ef_body, blk=EBLK),
        grid=(n_eblk,),
        in_specs=[pl.BlockSpec((EBLK, D_EDGE),
                               lambda i: (jnp.minimum(i, last_in), 0)),
                  pl.BlockSpec((EBLK, 1),
                               lambda i: (jnp.minimum(i, last_in), 0))],
        out_specs=pl.BlockSpec((EBLK, D_RBF + D_EDGE), lambda i: (i, 0)),
        out_shape=jax.ShapeDtypeStruct((E_PAD, D_RBF + D_EDGE), f32),
    )(edge_attribute, edge_length.reshape(E, 1))

    msg = pl.pallas_call(
        _msg_body,
        grid=(n_eblk,),
        in_specs=[pl.BlockSpec((EBLK, D_RBF + D_EDGE), lambda i: (i, 0)),
                  pl.BlockSpec((EBLK, F), lambda i: (i, 0)),
                  pl.BlockSpec((F, 16 * F), lambda i: (0, 0)),
                  pl.BlockSpec((F, 16 * F), lambda i: (0, 0))],
        out_specs=pl.BlockSpec((EBLK, F), lambda i: (i, 0)),
        out_shape=jax.ShapeDtypeStruct((E_PAD, F), f32),
    )

    gru = pl.pallas_call(
        _gru_body,
        out_shape=jax.ShapeDtypeStruct((N_PAD, F), f32),
    )

    sc_gather, sc_scatter = _build_sc_kernels()
    h = h0
    for _ in range(STEPS):
        hs = sc_gather(h, src)
        m = msg(ef, hs, bah, bal)
        aggp = sc_scatter(m, dst, zeros_np)
        h = gru(aggp, conv_b.reshape(1, F), h,
                gru_Wih, gru_Whh, gru_bih.reshape(1, 3 * F),
                gru_bhh.reshape(1, 3 * F))

    out = pl.pallas_call(
        _epi_body,
        out_shape=jax.ShapeDtypeStruct((G, D_HID), f32),
    )(h, h0, seg_pad, spars_W, spars_b.reshape(1, D_HID),
      prelu_a.reshape(1, 1))
    return out


# speedup vs baseline: 2.8521x; 1.2617x over previous
import functools

import jax
import jax.numpy as jnp
from jax import lax
from jax.experimental import pallas as pl
from jax.experimental.pallas import tpu as pltpu
from jax.experimental.pallas import tpu_sc as plsc

N = 10000
E = 160000
G = 64
F = 32
D_EDGE = 8
D_RBF = 8
D_HID = 4096
STEPS = 4

N_PAD = 10240
E_PAD = 163840
CHUNK = 128
N_CHUNKS = E_PAD // CHUNK
NW = 32
CH_PER_W = N_CHUNKS // NW
GRP = 8
N_GRP = CH_PER_W // GRP
ROWS_PER_SUB = N_PAD // 16

EBLK = 4096
HIGH = lax.Precision.HIGHEST

@functools.lru_cache(maxsize=1)
def _build_sc_kernels():
    mesh = plsc.VectorSubcoreMesh(core_axis_name="c", subcore_axis_name="s",
                                  num_cores=2, num_subcores=16)
    params = pltpu.CompilerParams(use_tc_tiling_on_sc=False)

    @functools.partial(
        pl.kernel,
        out_type=jax.ShapeDtypeStruct((E_PAD, 128), jnp.float32),
        mesh=mesh,
        compiler_params=params,
        scratch_types=[
            pltpu.VMEM((CH_PER_W, CHUNK), jnp.int32),
            pltpu.VMEM((2, GRP * CHUNK, F), jnp.float32),
            pltpu.SemaphoreType.DMA,
            pltpu.SemaphoreType.DMA,
            pltpu.SemaphoreType.DMA,
            pltpu.SemaphoreType.DMA,
        ],
    )
    def sc_gather(h_hbm, src_hbm, out_hbm, idx_v, rows_v, ga0, ga1, oa0, oa1):
        wid = lax.axis_index("s") * 2 + lax.axis_index("c")
        chunk0 = wid * CH_PER_W
        pltpu.sync_copy(src_hbm.at[pl.ds(chunk0, CH_PER_W)], idx_v)
        gsem = [ga0, ga1]
        osem = [oa0, oa1]
        gath = {}
        outc = {}

        def fire(g):
            b = g % 2
            gath[g] = [
                pltpu.async_copy(h_hbm.at[idx_v.at[g * GRP + j]],
                                 rows_v.at[b, pl.ds(j * CHUNK, CHUNK)],
                                 gsem[b])
                for j in range(GRP)
            ]

        fire(0)
        for g in range(N_GRP):
            b = g % 2
            if g + 1 < N_GRP:
                if g >= 1:
                    outc[g - 1].wait()
                fire(g + 1)
            for c in gath[g]:
                c.wait()
            outc[g] = pltpu.async_copy(
                rows_v.at[b],
                out_hbm.at[pl.ds((chunk0 + g * GRP) * CHUNK, GRP * CHUNK),
                           pl.ds(0, F)],
                osem[b])
        outc[N_GRP - 2].wait()
        outc[N_GRP - 1].wait()

    @functools.partial(
        pl.kernel,
        out_type=jax.ShapeDtypeStruct((2, N_PAD, F), jnp.float32),
        mesh=mesh,
        compiler_params=params,
        scratch_types=[
            pltpu.VMEM((CH_PER_W, CHUNK), jnp.int32),
            pltpu.VMEM((2, GRP * CHUNK, F), jnp.float32),
            pltpu.VMEM_SHARED((N_PAD, F), jnp.float32),
            pltpu.SemaphoreType.DMA,
            pltpu.SemaphoreType.DMA,
            pltpu.SemaphoreType.DMA,
            pltpu.SemaphoreType.DMA,
        ],
    )
    def sc_scatter(m_hbm, dst_hbm, zero_hbm, out_hbm, idx_v, rows_v, agg_sh,
                   ma0, ma1, sa0, sa1):
        cid = lax.axis_index("c")
        sid = lax.axis_index("s")
        wid = sid * 2 + cid
        row0 = sid * ROWS_PER_SUB
        chunk0 = wid * CH_PER_W
        zc = pltpu.async_copy(zero_hbm.at[pl.ds(row0, ROWS_PER_SUB)],
                              agg_sh.at[pl.ds(row0, ROWS_PER_SUB)], ma1)
        pltpu.sync_copy(dst_hbm.at[pl.ds(chunk0, CH_PER_W)], idx_v)
        zc.wait()
        plsc.subcore_barrier()
        msem = [ma0, ma1]
        ssem = [sa0, sa1]
        mload = {}
        adds = {}

        def fire_load(g):
            b = g % 2
            mload[g] = pltpu.async_copy(
                m_hbm.at[pl.ds((chunk0 + g * GRP) * CHUNK, GRP * CHUNK),
                         pl.ds(0, F)],
                rows_v.at[b], msem[b])

        fire_load(0)
        for g in range(N_GRP):
            b = g % 2
            mload[g].wait()
            if g + 1 < N_GRP:
                if g >= 1:
                    for c in adds[g - 1]:
                        c.wait()
                fire_load(g + 1)
            adds[g] = [
                pltpu.async_copy(rows_v.at[b, pl.ds(j * CHUNK, CHUNK)],
                                 agg_sh.at[idx_v.at[g * GRP + j]],
                                 ssem[b], add=True)
                for j in range(GRP)
            ]
        for g in (N_GRP - 2, N_GRP - 1):
            for c in adds[g]:
                c.wait()
        plsc.subcore_barrier()
        pltpu.sync_copy(agg_sh.at[pl.ds(row0, ROWS_PER_SUB)],
                        rows_v.at[0, pl.ds(0, ROWS_PER_SUB)])
        pltpu.sync_copy(rows_v.at[0, pl.ds(0, ROWS_PER_SUB)],
                        out_hbm.at[cid, pl.ds(row0, ROWS_PER_SUB)])

    return sc_gather, sc_scatter


def _proj_body(na_ref, w_ref, b_ref, out_ref):
    h = jnp.dot(na_ref[...], w_ref[...], preferred_element_type=jnp.float32,
                precision=HIGH)
    out_ref[pl.ds(0, N), :] = jnp.maximum(h + b_ref[...], 0.0)
    out_ref[pl.ds(N, N_PAD - N), :] = jnp.zeros((N_PAD - N, F), jnp.float32)


def _ef_body(ea_ref, el_ref, out_ref, *, blk):
    i = pl.program_id(0)
    el = el_ref[...]
    centers = lax.broadcasted_iota(jnp.int32, (1, D_RBF), 1).astype(jnp.float32) / (D_RBF - 1.0)
    rbf = jnp.exp(-10.0 * (el - centers) ** 2)
    ef = jnp.concatenate([ea_ref[...], rbf], axis=1)
    row = i * blk + lax.broadcasted_iota(jnp.int32, (blk, 1), 0)
    out_ref[...] = jnp.where(row < E, ef, 0.0)


def _bdot(a, b):
    return jax.lax.dot_general(a, b, (((1,), (0,)), ((), ())),
                               preferred_element_type=jnp.float32)


def _split(x):
    hi = x.astype(jnp.bfloat16)
    lo = (x - hi.astype(jnp.float32)).astype(jnp.bfloat16)
    return hi, lo


def _msg_body(ef_ref, hs_ref, bah_ref, bal_ref, out_ref):
    ef = ef_ref[...]
    hs = hs_ref[:, :F]
    hs_h, hs_l = _split(hs)
    bah = bah_ref[...]
    t = (_bdot(hs_h, bah) + _bdot(hs_l, bah)) + _bdot(hs_h, bal_ref[...])
    kidx = lax.broadcasted_iota(jnp.int32, (ef.shape[0], 16 * F), 1) // F
    ef_rep = jnp.take_along_axis(ef, kidx, axis=1)
    x = ef_rep * t
    x = x[:, :256] + x[:, 256:]
    x = x[:, :128] + x[:, 128:]
    x = x[:, :64] + x[:, 64:]
    out_ref[:, :F] = x[:, :F] + x[:, F:]


def _gru_body(agg_ref, cb_ref, hid_ref, wih_ref, whh_ref, bih_ref, bhh_ref,
              out_ref):
    x = jnp.maximum(agg_ref[0] + agg_ref[1] + cb_ref[...], 0.0)
    hid = hid_ref[...]
    gi = jnp.dot(x, wih_ref[...], preferred_element_type=jnp.float32,
                 precision=HIGH) + bih_ref[...]
    gh = jnp.dot(hid, whh_ref[...], preferred_element_type=jnp.float32,
                 precision=HIGH) + bhh_ref[...]
    r = jax.nn.sigmoid(gi[:, 0:F] + gh[:, 0:F])
    z = jax.nn.sigmoid(gi[:, F:2 * F] + gh[:, F:2 * F])
    n = jnp.tanh(gi[:, 2 * F:] + r * gh[:, 2 * F:])
    out_ref[...] = (1.0 - z) * n + z * hid


def _epi_body(h_ref, h0_ref, seg_ref, w_ref, b_ref, a_ref, out_ref):
    na = jnp.concatenate([h_ref[...], h0_ref[...]], axis=1)
    gids = lax.broadcasted_iota(jnp.int32, (1, G), 1)
    onehot = (seg_ref[...] == gids).astype(jnp.float32)
    r = lax.dot_general(onehot, na, (((0,), (0,)), ((), ())),
                        preferred_element_type=jnp.float32, precision=HIGH)
    y = jnp.dot(r, w_ref[...], preferred_element_type=jnp.float32,
                precision=HIGH) + b_ref[...]
    out_ref[...] = jnp.where(y >= 0.0, y, a_ref[0, 0] * y)


def kernel(node_attribute, edge_attribute, edge_length, edge_index, segment_ids,
           proj_W, proj_b, bond_W, bond_b, conv_b,
           gru_Wih, gru_Whh, gru_bih, gru_bhh,
           spars_W, spars_b, prelu_a):
    f32 = jnp.float32
    src = jnp.zeros((E_PAD,), jnp.int32).at[:E].set(edge_index[0]).reshape(N_CHUNKS, CHUNK)
    dst = jnp.zeros((E_PAD,), jnp.int32).at[:E].set(edge_index[1]).reshape(N_CHUNKS, CHUNK)
    seg_pad = jnp.full((N_PAD, 1), G, jnp.int32).at[:N, 0].set(segment_ids)
    b2all = bond_W.reshape(16, F, F).transpose(1, 0, 2).reshape(F, 16 * F)
    bah = b2all.astype(jnp.bfloat16)
    bal = (b2all - bah.astype(f32)).astype(jnp.bfloat16)
    zeros_np = jnp.zeros((N_PAD, F), f32)

    h0 = pl.pallas_call(
        _proj_body,
        out_shape=jax.ShapeDtypeStruct((N_PAD, F), f32),
    )(node_attribute, proj_W, proj_b.reshape(1, F))

    n_eblk = E_PAD // EBLK
    last_in = E // EBLK - (1 if E % EBLK == 0 else 0)
    ef = pl.pallas_call(
        functools.partial(_ef_body, blk=EBLK),
        grid=(n_eblk,),
        in_specs=[pl.BlockSpec((EBLK, D_EDGE),
                               lambda i: (jnp.minimum(i, last_in), 0)),
                  pl.BlockSpec((EBLK, 1),
                               lambda i: (jnp.minimum(i, last_in), 0))],
        out_specs=pl.BlockSpec((EBLK, D_RBF + D_EDGE), lambda i: (i, 0)),
        out_shape=jax.ShapeDtypeStruct((E_PAD, D_RBF + D_EDGE), f32),
    )(edge_attribute, edge_length.reshape(E, 1))

    msg = pl.pallas_call(
        _msg_body,
        grid=(n_eblk,),
        in_specs=[pl.BlockSpec((EBLK, D_RBF + D_EDGE), lambda i: (i, 0)),
                  pl.BlockSpec((EBLK, 128), lambda i: (i, 0)),
                  pl.BlockSpec((F, 16 * F), lambda i: (0, 0)),
                  pl.BlockSpec((F, 16 * F), lambda i: (0, 0))],
        out_specs=pl.BlockSpec((EBLK, 128), lambda i: (i, 0)),
        out_shape=jax.ShapeDtypeStruct((E_PAD, 128), f32),
    )

    gru = pl.pallas_call(
        _gru_body,
        out_shape=jax.ShapeDtypeStruct((N_PAD, F), f32),
    )

    sc_gather, sc_scatter = _build_sc_kernels()
    h = h0
    for _ in range(STEPS):
        hs = sc_gather(h, src)
        m = msg(ef, hs, bah, bal)
        aggp = sc_scatter(m, dst, zeros_np)
        h = gru(aggp, conv_b.reshape(1, F), h,
                gru_Wih, gru_Whh, gru_bih.reshape(1, 3 * F),
                gru_bhh.reshape(1, 3 * F))

    out = pl.pallas_call(
        _epi_body,
        out_shape=jax.ShapeDtypeStruct((G, D_HID), f32),
    )(h, h0, seg_pad, spars_W, spars_b.reshape(1, D_HID),
      prelu_a.reshape(1, 1))
    return out


# speedup vs baseline: 2.8761x; 1.0084x over previous
import functools

import jax
import jax.numpy as jnp
from jax import lax
from jax.experimental import pallas as pl
from jax.experimental.pallas import tpu as pltpu
from jax.experimental.pallas import tpu_sc as plsc

N = 10000
E = 160000
G = 64
F = 32
D_EDGE = 8
D_RBF = 8
D_HID = 4096
STEPS = 4

N_PAD = 10240
E_PAD = 163840
CHUNK = 128
N_CHUNKS = E_PAD // CHUNK
NW = 32
CH_PER_W = N_CHUNKS // NW
GRP = 8
N_GRP = CH_PER_W // GRP
ROWS_PER_SUB = N_PAD // 16

EBLK = 4096
HIGH = lax.Precision.HIGHEST

@functools.lru_cache(maxsize=1)
def _build_sc_kernels():
    mesh = plsc.VectorSubcoreMesh(core_axis_name="c", subcore_axis_name="s",
                                  num_cores=2, num_subcores=16)
    params = pltpu.CompilerParams(use_tc_tiling_on_sc=False)

    @functools.partial(
        pl.kernel,
        out_type=jax.ShapeDtypeStruct((E_PAD, 128), jnp.float32),
        mesh=mesh,
        compiler_params=params,
        scratch_types=[
            pltpu.VMEM((CH_PER_W, CHUNK), jnp.int32),
            pltpu.VMEM((2, GRP * CHUNK, F), jnp.float32),
            pltpu.SemaphoreType.DMA,
            pltpu.SemaphoreType.DMA,
            pltpu.SemaphoreType.DMA,
            pltpu.SemaphoreType.DMA,
        ],
    )
    def sc_gather(h_hbm, src_hbm, out_hbm, idx_v, rows_v, ga0, ga1, oa0, oa1):
        wid = lax.axis_index("s") * 2 + lax.axis_index("c")
        chunk0 = wid * CH_PER_W
        pltpu.sync_copy(src_hbm.at[pl.ds(chunk0, CH_PER_W)], idx_v)
        gsem = [ga0, ga1]
        osem = [oa0, oa1]
        gath = {}
        outc = {}

        def fire(g):
            b = g % 2
            gath[g] = [
                pltpu.async_copy(h_hbm.at[idx_v.at[g * GRP + j]],
                                 rows_v.at[b, pl.ds(j * CHUNK, CHUNK)],
                                 gsem[b])
                for j in range(GRP)
            ]

        fire(0)
        for g in range(N_GRP):
            b = g % 2
            if g + 1 < N_GRP:
                if g >= 1:
                    outc[g - 1].wait()
                fire(g + 1)
            for c in gath[g]:
                c.wait()
            outc[g] = pltpu.async_copy(
                rows_v.at[b],
                out_hbm.at[pl.ds((chunk0 + g * GRP) * CHUNK, GRP * CHUNK),
                           pl.ds(0, F)],
                osem[b])
        outc[N_GRP - 2].wait()
        outc[N_GRP - 1].wait()

    @functools.partial(
        pl.kernel,
        out_type=jax.ShapeDtypeStruct((2, N_PAD, F), jnp.float32),
        mesh=mesh,
        compiler_params=params,
        scratch_types=[
            pltpu.VMEM((CH_PER_W, CHUNK), jnp.int32),
            pltpu.VMEM((2, GRP * CHUNK, F), jnp.float32),
            pltpu.VMEM_SHARED((N_PAD, F), jnp.float32),
            pltpu.SemaphoreType.DMA,
            pltpu.SemaphoreType.DMA,
            pltpu.SemaphoreType.DMA,
            pltpu.SemaphoreType.DMA,
        ],
    )
    def sc_scatter(m_hbm, dst_hbm, zero_hbm, out_hbm, idx_v, rows_v, agg_sh,
                   ma0, ma1, sa0, sa1):
        cid = lax.axis_index("c")
        sid = lax.axis_index("s")
        wid = sid * 2 + cid
        row0 = sid * ROWS_PER_SUB
        chunk0 = wid * CH_PER_W
        zc = pltpu.async_copy(zero_hbm.at[pl.ds(row0, ROWS_PER_SUB)],
                              agg_sh.at[pl.ds(row0, ROWS_PER_SUB)], ma1)
        pltpu.sync_copy(dst_hbm.at[pl.ds(chunk0, CH_PER_W)], idx_v)
        zc.wait()
        plsc.subcore_barrier()
        msem = [ma0, ma1]
        ssem = [sa0, sa1]
        mload = {}
        adds = {}

        def fire_load(g):
            b = g % 2
            mload[g] = pltpu.async_copy(
                m_hbm.at[pl.ds((chunk0 + g * GRP) * CHUNK, GRP * CHUNK),
                         pl.ds(0, F)],
                rows_v.at[b], msem[b])

        fire_load(0)
        for g in range(N_GRP):
            b = g % 2
            mload[g].wait()
            if g + 1 < N_GRP:
                if g >= 1:
                    for c in adds[g - 1]:
                        c.wait()
                fire_load(g + 1)
            adds[g] = [
                pltpu.async_copy(rows_v.at[b, pl.ds(j * CHUNK, CHUNK)],
                                 agg_sh.at[idx_v.at[g * GRP + j]],
                                 ssem[b], add=True)
                for j in range(GRP)
            ]
        for g in (N_GRP - 2, N_GRP - 1):
            for c in adds[g]:
                c.wait()
        plsc.subcore_barrier()
        pltpu.sync_copy(agg_sh.at[pl.ds(row0, ROWS_PER_SUB)],
                        rows_v.at[0, pl.ds(0, ROWS_PER_SUB)])
        pltpu.sync_copy(rows_v.at[0, pl.ds(0, ROWS_PER_SUB)],
                        out_hbm.at[cid, pl.ds(row0, ROWS_PER_SUB)])

    return sc_gather, sc_scatter


def _dot3(a, b):
    a_h, a_l = _split(a)
    b_h, b_l = _split(b)
    return (_bdot(a_h, b_h) + _bdot(a_l, b_h)) + _bdot(a_h, b_l)


def _proj_body(na_ref, w_ref, b_ref, out_ref):
    h = _dot3(na_ref[...], w_ref[...])
    out_ref[pl.ds(0, N), :] = jnp.maximum(h + b_ref[...], 0.0)
    out_ref[pl.ds(N, N_PAD - N), :] = jnp.zeros((N_PAD - N, F), jnp.float32)


def _ef_body(ea_ref, el_ref, out_ref, *, blk):
    i = pl.program_id(0)
    el = el_ref[...]
    centers = lax.broadcasted_iota(jnp.int32, (1, D_RBF), 1).astype(jnp.float32) / (D_RBF - 1.0)
    rbf = jnp.exp(-10.0 * (el - centers) ** 2)
    ef = jnp.concatenate([ea_ref[...], rbf], axis=1)
    row = i * blk + lax.broadcasted_iota(jnp.int32, (blk, 1), 0)
    out_ref[...] = jnp.where(row < E, ef, 0.0)


def _bdot(a, b):
    return jax.lax.dot_general(a, b, (((1,), (0,)), ((), ())),
                               preferred_element_type=jnp.float32)


def _split(x):
    hi = x.astype(jnp.bfloat16)
    lo = (x - hi.astype(jnp.float32)).astype(jnp.bfloat16)
    return hi, lo


def _msg_body(ef_ref, hs_ref, bah_ref, bal_ref, out_ref):
    ef = ef_ref[...]
    hs = hs_ref[:, :F]
    hs_h, hs_l = _split(hs)
    bah = bah_ref[...]
    t = (_bdot(hs_h, bah) + _bdot(hs_l, bah)) + _bdot(hs_h, bal_ref[...])
    kidx = lax.broadcasted_iota(jnp.int32, (ef.shape[0], 16 * F), 1) // F
    ef_rep = jnp.take_along_axis(ef, kidx, axis=1)
    x = ef_rep * t
    x = x[:, :256] + x[:, 256:]
    x = x[:, :128] + x[:, 128:]
    x = x[:, :64] + x[:, 64:]
    out_ref[:, :F] = x[:, :F] + x[:, F:]


def _gru_body(agg_ref, cb_ref, hid_ref, wih_ref, whh_ref, bih_ref, bhh_ref,
              out_ref):
    x = jnp.maximum(agg_ref[0] + agg_ref[1] + cb_ref[...], 0.0)
    hid = hid_ref[...]
    gi = _dot3(x, wih_ref[...]) + bih_ref[...]
    gh = _dot3(hid, whh_ref[...]) + bhh_ref[...]
    r = jax.nn.sigmoid(gi[:, 0:F] + gh[:, 0:F])
    z = jax.nn.sigmoid(gi[:, F:2 * F] + gh[:, F:2 * F])
    n = jnp.tanh(gi[:, 2 * F:] + r * gh[:, 2 * F:])
    out_ref[...] = (1.0 - z) * n + z * hid


def _epi_body(h_ref, h0_ref, seg_ref, w_ref, b_ref, a_ref, out_ref):
    na = jnp.concatenate([h_ref[...], h0_ref[...]], axis=1)
    gids = lax.broadcasted_iota(jnp.int32, (1, G), 1)
    onehot = (seg_ref[...] == gids).astype(jnp.bfloat16)
    na_h, na_l = _split(na)
    dot0 = lambda a, b: lax.dot_general(
        a, b, (((0,), (0,)), ((), ())), preferred_element_type=jnp.float32)
    r = dot0(onehot, na_h) + dot0(onehot, na_l)
    y = _dot3(r, w_ref[...]) + b_ref[...]
    out_ref[...] = jnp.where(y >= 0.0, y, a_ref[0, 0] * y)


def kernel(node_attribute, edge_attribute, edge_length, edge_index, segment_ids,
           proj_W, proj_b, bond_W, bond_b, conv_b,
           gru_Wih, gru_Whh, gru_bih, gru_bhh,
           spars_W, spars_b, prelu_a):
    f32 = jnp.float32
    src = jnp.zeros((E_PAD,), jnp.int32).at[:E].set(edge_index[0]).reshape(N_CHUNKS, CHUNK)
    dst = jnp.zeros((E_PAD,), jnp.int32).at[:E].set(edge_index[1]).reshape(N_CHUNKS, CHUNK)
    seg_pad = jnp.full((N_PAD, 1), G, jnp.int32).at[:N, 0].set(segment_ids)
    b2all = bond_W.reshape(16, F, F).transpose(1, 0, 2).reshape(F, 16 * F)
    bah = b2all.astype(jnp.bfloat16)
    bal = (b2all - bah.astype(f32)).astype(jnp.bfloat16)
    zeros_np = jnp.zeros((N_PAD, F), f32)

    h0 = pl.pallas_call(
        _proj_body,
        out_shape=jax.ShapeDtypeStruct((N_PAD, F), f32),
    )(node_attribute, proj_W, proj_b.reshape(1, F))

    n_eblk = E_PAD // EBLK
    last_in = E // EBLK - (1 if E % EBLK == 0 else 0)
    ef = pl.pallas_call(
        functools.partial(_ef_body, blk=EBLK),
        grid=(n_eblk,),
        in_specs=[pl.BlockSpec((EBLK, D_EDGE),
                               lambda i: (jnp.minimum(i, last_in), 0)),
                  pl.BlockSpec((EBLK, 1),
                               lambda i: (jnp.minimum(i, last_in), 0))],
        out_specs=pl.BlockSpec((EBLK, D_RBF + D_EDGE), lambda i: (i, 0)),
        out_shape=jax.ShapeDtypeStruct((E_PAD, D_RBF + D_EDGE), f32),
    )(edge_attribute, edge_length.reshape(E, 1))

    msg = pl.pallas_call(
        _msg_body,
        grid=(n_eblk,),
        in_specs=[pl.BlockSpec((EBLK, D_RBF + D_EDGE), lambda i: (i, 0)),
                  pl.BlockSpec((EBLK, 128), lambda i: (i, 0)),
                  pl.BlockSpec((F, 16 * F), lambda i: (0, 0)),
                  pl.BlockSpec((F, 16 * F), lambda i: (0, 0))],
        out_specs=pl.BlockSpec((EBLK, 128), lambda i: (i, 0)),
        out_shape=jax.ShapeDtypeStruct((E_PAD, 128), f32),
    )

    gru = pl.pallas_call(
        _gru_body,
        out_shape=jax.ShapeDtypeStruct((N_PAD, F), f32),
    )

    sc_gather, sc_scatter = _build_sc_kernels()
    h = h0
    for _ in range(STEPS):
        hs = sc_gather(h, src)
        m = msg(ef, hs, bah, bal)
        aggp = sc_scatter(m, dst, zeros_np)
        h = gru(aggp, conv_b.reshape(1, F), h,
                gru_Wih, gru_Whh, gru_bih.reshape(1, 3 * F),
                gru_bhh.reshape(1, 3 * F))

    out = pl.pallas_call(
        _epi_body,
        out_shape=jax.ShapeDtypeStruct((G, D_HID), f32),
    )(h, h0, seg_pad, spars_W, spars_b.reshape(1, D_HID),
      prelu_a.reshape(1, 1))
    return out
